# Initial kernel scaffold; baseline (speedup 1.0000x reference)
import functools
import jax
import jax.numpy as jnp
from jax import lax
from jax.experimental import pallas as pl
from jax.experimental.pallas import tpu as pltpu
from jax.experimental.pallas import tpu_sc as plsc

N = 10000
E = 320000
F = 128
NG = 50
CUTOFF = 5.0
_WIDTH = CUTOFF / (NG - 1)
_COEFF = -0.5 / (_WIDTH * _WIDTH)
_LOG2 = 0.6931471805599453

NC, NS = 2, 16
NW = NC * NS
EPW = E // NW
EC = 80
NECH = EPW // EC
NPAD = 10240
ZPW = NPAD // NW
ZC = 64
NZCH = ZPW // ZC
RPT = N // NS
RC = 125
NRCH = RPT // RC

_mesh = functools.partial(plsc.VectorSubcoreMesh,
                          core_axis_name="c", subcore_axis_name="s")


def _prep_body(xyzt, a0h, a1h, zh, embh, d2h, s0h,
               xc, yc, zc, a0v, a1v, d2v, zidx, erows, sem):
    cid = lax.axis_index("c")
    sid = lax.axis_index("s")
    wid = sid * NC + cid

    pltpu.sync_copy(xyzt.at[0], xc)
    pltpu.sync_copy(xyzt.at[1], yc)
    pltpu.sync_copy(xyzt.at[2], zc)
    ebase = wid * EPW
    pltpu.sync_copy(a0h.at[pl.ds(ebase, EPW)], a0v)
    pltpu.sync_copy(a1h.at[pl.ds(ebase, EPW)], a1v)

    def dist_step(i, carry):
        i0 = a0v[pl.ds(i * 16, 16)]
        i1 = a1v[pl.ds(i * 16, 16)]
        dx = plsc.load_gather(xc, [i0]) - plsc.load_gather(xc, [i1])
        dy = plsc.load_gather(yc, [i0]) - plsc.load_gather(yc, [i1])
        dz = plsc.load_gather(zc, [i0]) - plsc.load_gather(zc, [i1])
        d2v[pl.ds(i * 16, 16)] = dx * dx + dy * dy + dz * dz
        return carry

    lax.fori_loop(0, EPW // 16, dist_step, 0)
    pltpu.sync_copy(d2v, d2h.at[pl.ds(ebase, EPW)])

    nbase = wid * ZPW

    def emb_step(k, carry):
        off = nbase + k * ZC
        pltpu.sync_copy(zh.at[pl.ds(off, ZC)], zidx)
        pltpu.async_copy(embh.at[zidx], erows, sem).wait()
        pltpu.sync_copy(erows, s0h.at[pl.ds(off, ZC)])
        return carry

    lax.fori_loop(0, NZCH, emb_step, 0)


@jax.jit
def _prep(xyzt, a0, a1, zp, embed):
    return pl.kernel(
        _prep_body,
        out_type=(jax.ShapeDtypeStruct((E,), jnp.float32),
                  jax.ShapeDtypeStruct((NPAD, F), jnp.float32)),
        mesh=_mesh(),
        scratch_types=[
            pltpu.VMEM((N,), jnp.float32),
            pltpu.VMEM((N,), jnp.float32),
            pltpu.VMEM((N,), jnp.float32),
            pltpu.VMEM((EPW,), jnp.int32),
            pltpu.VMEM((EPW,), jnp.int32),
            pltpu.VMEM((EPW,), jnp.float32),
            pltpu.VMEM((ZC,), jnp.int32),
            pltpu.VMEM((ZC, F), jnp.float32),
            pltpu.SemaphoreType.DMA,
        ],
    )(xyzt, a0, a1, zp, embed)


def _msg_body(rnh, wh, a0h, a1h, aggh,
              aggs, a0v, a1v, r0, r1, wv, vbuf, sem):
    cid = lax.axis_index("c")
    sid = lax.axis_index("s")
    wid = sid * NC + cid
    zeros = jnp.zeros((16,), jnp.float32)

    def zrow(i, carry):
        for j in range(F // 16):
            vbuf[i, pl.ds(j * 16, 16)] = zeros
        return carry

    lax.fori_loop(0, RC, zrow, 0)

    rowbase = sid * RPT

    def zchunk(k, carry):
        pltpu.sync_copy(vbuf, aggs.at[pl.ds(rowbase + k * RC, RC)])
        return carry

    lax.fori_loop(0, NRCH, zchunk, 0)
    plsc.subcore_barrier()

    ebase = wid * EPW

    def echunk(g, carry):
        off = ebase + g * EC
        pltpu.sync_copy(a0h.at[pl.ds(off, EC)], a0v)
        pltpu.sync_copy(a1h.at[pl.ds(off, EC)], a1v)
        pltpu.async_copy(rnh.at[a0v], r0, sem).wait()
        pltpu.async_copy(rnh.at[a1v], r1, sem).wait()
        pltpu.sync_copy(wh.at[pl.ds(off, EC)], wv)

        def mul(i, c2):
            for j in range(F // 16):
                sl = pl.ds(j * 16, 16)
                ww = wv[i, sl]
                r0[i, sl] = r0[i, sl] * ww
                r1[i, sl] = r1[i, sl] * ww
            return c2

        lax.fori_loop(0, EC, mul, 0)
        pltpu.sync_copy(r0, aggs.at[a1v], add=True)
        pltpu.sync_copy(r1, aggs.at[a0v], add=True)
        return carry

    lax.fori_loop(0, NECH, echunk, 0)
    plsc.subcore_barrier()

    def ochunk(k, carry):
        r = rowbase + k * RC
        pltpu.sync_copy(aggs.at[pl.ds(r, RC)], vbuf)
        pltpu.sync_copy(vbuf, aggh.at[pl.ds(cid * N + r, RC)])
        return carry

    lax.fori_loop(0, NRCH, ochunk, 0)


@jax.jit
def _msg(rn, w, a0, a1):
    return pl.kernel(
        _msg_body,
        out_type=jax.ShapeDtypeStruct((2 * N, F), jnp.float32),
        mesh=_mesh(),
        scratch_types=[
            pltpu.VMEM_SHARED((N, F), jnp.float32),
            pltpu.VMEM((EC,), jnp.int32),
            pltpu.VMEM((EC,), jnp.int32),
            pltpu.VMEM((EC, F), jnp.float32),
            pltpu.VMEM((EC, F), jnp.float32),
            pltpu.VMEM((EC, F), jnp.float32),
            pltpu.VMEM((RC, F), jnp.float32),
            pltpu.SemaphoreType.DMA,
        ],
    )(rn, w, a0, a1)


def _ssp(x):
    return jax.nn.softplus(x) - _LOG2


def _edge_filter_body(d2_ref, we1, be1, we2, be2, w_ref):
    e = jnp.sqrt(d2_ref[:])
    eb = e.shape[0]
    offs = lax.broadcasted_iota(jnp.float32, (1, NG), 1) * _WIDTH
    diff = jnp.broadcast_to(e, (eb, NG)) - offs
    g = jnp.exp(_COEFF * diff * diff)
    h = _ssp(jnp.dot(g, we1[:], preferred_element_type=jnp.float32) + be1[:])
    w_ref[:] = jnp.dot(h, we2[:], preferred_element_type=jnp.float32) + be2[:]


_EB = 2000


@jax.jit
def _edge_filter(d2, we1, be1, we2, be2):
    return pl.pallas_call(
        _edge_filter_body,
        grid=(E // _EB,),
        in_specs=[
            pl.BlockSpec((_EB, 1), lambda i: (i, 0)),
            pl.BlockSpec((NG, NG), lambda i: (0, 0)),
            pl.BlockSpec((1, NG), lambda i: (0, 0)),
            pl.BlockSpec((NG, F), lambda i: (0, 0)),
            pl.BlockSpec((1, F), lambda i: (0, 0)),
        ],
        out_specs=pl.BlockSpec((_EB, F), lambda i: (i, 0)),
        out_shape=jax.ShapeDtypeStruct((E, F), jnp.float32),
        compiler_params=pltpu.CompilerParams(
            dimension_semantics=("arbitrary",)),
    )(d2, we1, be1, we2, be2)


def _rn_body(s_ref, wn, bn, rn_ref):
    rn_ref[:] = jnp.dot(s_ref[:], wn[:],
                        preferred_element_type=jnp.float32) + bn[:]


_NB = 2000


@jax.jit
def _rn(s, wn, bn):
    return pl.pallas_call(
        _rn_body,
        grid=(N // _NB,),
        in_specs=[
            pl.BlockSpec((_NB, F), lambda i: (i, 0)),
            pl.BlockSpec((F, F), lambda i: (0, 0)),
            pl.BlockSpec((1, F), lambda i: (0, 0)),
        ],
        out_specs=pl.BlockSpec((_NB, F), lambda i: (i, 0)),
        out_shape=jax.ShapeDtypeStruct((N, F), jnp.float32),
        compiler_params=pltpu.CompilerParams(
            dimension_semantics=("arbitrary",)),
    )(s, wn, bn)


def _update_body(a0_ref, a1_ref, s_ref, wu1, bu1, wu2, bu2, out_ref):
    agg = a0_ref[:] + a1_ref[:]
    u = _ssp(jnp.dot(agg, wu1[:], preferred_element_type=jnp.float32) + bu1[:])
    u = jnp.dot(u, wu2[:], preferred_element_type=jnp.float32) + bu2[:]
    out_ref[:] = s_ref[:] + u


@jax.jit
def _update(agg0, agg1, s, wu1, bu1, wu2, bu2):
    return pl.pallas_call(
        _update_body,
        grid=(N // _NB,),
        in_specs=[
            pl.BlockSpec((_NB, F), lambda i: (i, 0)),
            pl.BlockSpec((_NB, F), lambda i: (i, 0)),
            pl.BlockSpec((_NB, F), lambda i: (i, 0)),
            pl.BlockSpec((F, F), lambda i: (0, 0)),
            pl.BlockSpec((1, F), lambda i: (0, 0)),
            pl.BlockSpec((F, F), lambda i: (0, 0)),
            pl.BlockSpec((1, F), lambda i: (0, 0)),
        ],
        out_specs=pl.BlockSpec((_NB, F), lambda i: (i, 0)),
        out_shape=jax.ShapeDtypeStruct((N, F), jnp.float32),
        compiler_params=pltpu.CompilerParams(
            dimension_semantics=("arbitrary",)),
    )(agg0, agg1, s, wu1, bu1, wu2, bu2)


def kernel(z, xyz, nbr_list, embed, params):
    a0 = jnp.ascontiguousarray(nbr_list[:, 0])
    a1 = jnp.ascontiguousarray(nbr_list[:, 1])
    xyzt = jnp.ascontiguousarray(xyz.T)
    zp = jnp.pad(z, (0, NPAD - N))
    d2, s0p = _prep(xyzt, a0, a1, zp, embed)
    s = s0p[:N]
    d2 = d2[:, None]
    for p in params:
        w = _edge_filter(d2, p['We1'], p['be1'][None, :],
                         p['We2'], p['be2'][None, :])
        rn = _rn(s, p['Wn'], p['bn'][None, :])
        agg = _msg(rn, w, a0, a1)
        s = _update(agg[:N], agg[N:], s,
                    p['Wu1'], p['bu1'][None, :],
                    p['Wu2'], p['bu2'][None, :])
    return s


# speedup vs baseline: 3.8161x; 3.8161x over previous
import functools
import jax
import jax.numpy as jnp
from jax import lax
from jax.experimental import pallas as pl
from jax.experimental.pallas import tpu as pltpu
from jax.experimental.pallas import tpu_sc as plsc

N = 10000
E = 320000
F = 128
NG = 50
CUTOFF = 5.0
_WIDTH = CUTOFF / (NG - 1)
_COEFF = -0.5 / (_WIDTH * _WIDTH)
_LOG2 = 0.6931471805599453

NC, NS = 2, 16
NW = NC * NS
EPW = E // NW
EC = 80
NECH = EPW // EC
NPAD = 10240
ZPW = NPAD // NW
ZC = 64
NZCH = ZPW // ZC
RC = 80
NRCH = N // RC

_mesh = functools.partial(plsc.VectorSubcoreMesh,
                          core_axis_name="c", subcore_axis_name="s")
_SC_PARAMS = pltpu.CompilerParams(needs_layout_passes=False)


def _prep_body(xh, yh, zzh, a0h, a1h, zh, embh, d2h, s0h,
               xc, yc, zc, a0v, a1v, d2v, zidx, erows, sem):
    cid = lax.axis_index("c")
    sid = lax.axis_index("s")
    wid = sid * NC + cid

    pltpu.sync_copy(xh, xc)
    pltpu.sync_copy(yh, yc)
    pltpu.sync_copy(zzh, zc)
    ebase = wid * EPW
    pltpu.sync_copy(a0h.at[pl.ds(ebase, EPW)], a0v)
    pltpu.sync_copy(a1h.at[pl.ds(ebase, EPW)], a1v)

    def dist_step(i, carry):
        i0 = a0v[pl.ds(i * 16, 16)]
        i1 = a1v[pl.ds(i * 16, 16)]
        dx = plsc.load_gather(xc, [i0]) - plsc.load_gather(xc, [i1])
        dy = plsc.load_gather(yc, [i0]) - plsc.load_gather(yc, [i1])
        dz = plsc.load_gather(zc, [i0]) - plsc.load_gather(zc, [i1])
        d2v[pl.ds(i * 16, 16)] = dx * dx + dy * dy + dz * dz
        return carry

    lax.fori_loop(0, EPW // 16, dist_step, 0)
    pltpu.sync_copy(d2v, d2h.at[pl.ds(ebase, EPW)])

    nbase = wid * ZPW

    def emb_step(k, carry):
        off = nbase + k * ZC
        pltpu.sync_copy(zh.at[pl.ds(off, ZC)], zidx)
        pltpu.async_copy(embh.at[zidx], erows, sem).wait()
        pltpu.sync_copy(erows, s0h.at[pl.ds(off, ZC)])
        return carry

    lax.fori_loop(0, NZCH, emb_step, 0)


@jax.jit
def _prep(xcol, ycol, zcol, a0, a1, zp, embed):
    return pl.kernel(
        _prep_body,
        out_type=(jax.ShapeDtypeStruct((E,), jnp.float32),
                  jax.ShapeDtypeStruct((NPAD, F), jnp.float32)),
        mesh=_mesh(),
        scratch_types=[
            pltpu.VMEM((N,), jnp.float32),
            pltpu.VMEM((N,), jnp.float32),
            pltpu.VMEM((N,), jnp.float32),
            pltpu.VMEM((EPW,), jnp.int32),
            pltpu.VMEM((EPW,), jnp.int32),
            pltpu.VMEM((EPW,), jnp.float32),
            pltpu.VMEM((ZC,), jnp.int32),
            pltpu.VMEM((ZC, F), jnp.float32),
            pltpu.SemaphoreType.DMA,
        ],
        compiler_params=_SC_PARAMS,
    )(xcol, ycol, zcol, a0, a1, zp, embed)


def _msg_body(rnh, wh, a0h, a1h, aggh,
              aggs, a0v, a1v, r0, r1, wv, vbuf, sem):
    cid = lax.axis_index("c")
    sid = lax.axis_index("s")
    wid = sid * NC + cid
    zeros = jnp.zeros((16,), jnp.float32)

    def zrow(i, carry):
        for j in range(F // 16):
            vbuf[i, pl.ds(j * 16, 16)] = zeros
        return carry

    lax.fori_loop(0, RC, zrow, 0)

    def zchunk(i, carry):
        k = sid + i * NS

        @pl.when(k < NRCH)
        def _():
            pltpu.sync_copy(vbuf, aggs.at[pl.ds(k * RC, RC)])

        return carry

    lax.fori_loop(0, (NRCH + NS - 1) // NS, zchunk, 0)
    plsc.subcore_barrier()

    ebase = wid * EPW

    def echunk(g, carry):
        off = ebase + g * EC
        pltpu.sync_copy(a0h.at[pl.ds(off, EC)], a0v)
        pltpu.sync_copy(a1h.at[pl.ds(off, EC)], a1v)
        pltpu.async_copy(rnh.at[a0v], r0, sem).wait()
        pltpu.async_copy(rnh.at[a1v], r1, sem).wait()
        pltpu.sync_copy(wh.at[pl.ds(off, EC)], wv)

        def mul(i, c2):
            for j in range(F // 16):
                sl = pl.ds(j * 16, 16)
                ww = wv[i, sl]
                r0[i, sl] = r0[i, sl] * ww
                r1[i, sl] = r1[i, sl] * ww
            return c2

        lax.fori_loop(0, EC, mul, 0)
        pltpu.sync_copy(r0, aggs.at[a1v], add=True)
        pltpu.sync_copy(r1, aggs.at[a0v], add=True)
        return carry

    lax.fori_loop(0, NECH, echunk, 0)
    plsc.subcore_barrier()

    def ochunk(i, carry):
        k = sid + i * NS

        @pl.when(k < NRCH)
        def _():
            r = k * RC
            pltpu.sync_copy(aggs.at[pl.ds(r, RC)], vbuf)
            pltpu.sync_copy(vbuf, aggh.at[pl.ds(cid * N + r, RC)])

        return carry

    lax.fori_loop(0, (NRCH + NS - 1) // NS, ochunk, 0)


@jax.jit
def _msg(rn, w, a0, a1):
    return pl.kernel(
        _msg_body,
        out_type=jax.ShapeDtypeStruct((2 * N, F), jnp.float32),
        mesh=_mesh(),
        scratch_types=[
            pltpu.VMEM_SHARED((N, F), jnp.float32),
            pltpu.VMEM((EC,), jnp.int32),
            pltpu.VMEM((EC,), jnp.int32),
            pltpu.VMEM((EC, F), jnp.float32),
            pltpu.VMEM((EC, F), jnp.float32),
            pltpu.VMEM((EC, F), jnp.float32),
            pltpu.VMEM((RC, F), jnp.float32),
            pltpu.SemaphoreType.DMA,
        ],
        compiler_params=_SC_PARAMS,
    )(rn, w, a0, a1)


def _ssp(x):
    return jax.nn.softplus(x) - _LOG2


def _edge_filter_body(d2_ref, we1, be1, we2, be2, w_ref):
    e = jnp.sqrt(d2_ref[:])
    eb = e.shape[0]
    offs = lax.broadcasted_iota(jnp.int32, (1, NG), 1).astype(jnp.float32) * _WIDTH
    diff = jnp.broadcast_to(e, (eb, NG)) - offs
    g = jnp.exp(_COEFF * diff * diff)
    h = _ssp(jnp.dot(g, we1[:], preferred_element_type=jnp.float32) + be1[:])
    w_ref[:] = jnp.dot(h, we2[:], preferred_element_type=jnp.float32) + be2[:]


_EB = 2000


@jax.jit
def _edge_filter(d2, we1, be1, we2, be2):
    return pl.pallas_call(
        _edge_filter_body,
        grid=(E // _EB,),
        in_specs=[
            pl.BlockSpec((_EB, 1), lambda i: (i, 0)),
            pl.BlockSpec((NG, NG), lambda i: (0, 0)),
            pl.BlockSpec((1, NG), lambda i: (0, 0)),
            pl.BlockSpec((NG, F), lambda i: (0, 0)),
            pl.BlockSpec((1, F), lambda i: (0, 0)),
        ],
        out_specs=pl.BlockSpec((_EB, F), lambda i: (i, 0)),
        out_shape=jax.ShapeDtypeStruct((E, F), jnp.float32),
        compiler_params=pltpu.CompilerParams(
            dimension_semantics=("arbitrary",)),
    )(d2, we1, be1, we2, be2)


def _rn_body(s_ref, wn, bn, rn_ref):
    rn_ref[:] = jnp.dot(s_ref[:], wn[:],
                        preferred_element_type=jnp.float32) + bn[:]


_NB = 2000


@jax.jit
def _rn(s, wn, bn):
    return pl.pallas_call(
        _rn_body,
        grid=(N // _NB,),
        in_specs=[
            pl.BlockSpec((_NB, F), lambda i: (i, 0)),
            pl.BlockSpec((F, F), lambda i: (0, 0)),
            pl.BlockSpec((1, F), lambda i: (0, 0)),
        ],
        out_specs=pl.BlockSpec((_NB, F), lambda i: (i, 0)),
        out_shape=jax.ShapeDtypeStruct((N, F), jnp.float32),
        compiler_params=pltpu.CompilerParams(
            dimension_semantics=("arbitrary",)),
    )(s, wn, bn)


def _update_body(a0_ref, a1_ref, s_ref, wu1, bu1, wu2, bu2, out_ref):
    agg = a0_ref[:] + a1_ref[:]
    u = _ssp(jnp.dot(agg, wu1[:], preferred_element_type=jnp.float32) + bu1[:])
    u = jnp.dot(u, wu2[:], preferred_element_type=jnp.float32) + bu2[:]
    out_ref[:] = s_ref[:] + u


@jax.jit
def _update(agg0, agg1, s, wu1, bu1, wu2, bu2):
    return pl.pallas_call(
        _update_body,
        grid=(N // _NB,),
        in_specs=[
            pl.BlockSpec((_NB, F), lambda i: (i, 0)),
            pl.BlockSpec((_NB, F), lambda i: (i, 0)),
            pl.BlockSpec((_NB, F), lambda i: (i, 0)),
            pl.BlockSpec((F, F), lambda i: (0, 0)),
            pl.BlockSpec((1, F), lambda i: (0, 0)),
            pl.BlockSpec((F, F), lambda i: (0, 0)),
            pl.BlockSpec((1, F), lambda i: (0, 0)),
        ],
        out_specs=pl.BlockSpec((_NB, F), lambda i: (i, 0)),
        out_shape=jax.ShapeDtypeStruct((N, F), jnp.float32),
        compiler_params=pltpu.CompilerParams(
            dimension_semantics=("arbitrary",)),
    )(agg0, agg1, s, wu1, bu1, wu2, bu2)


def kernel(z, xyz, nbr_list, embed, params):
    a0 = nbr_list[:, 0]
    a1 = nbr_list[:, 1]
    zp = jnp.pad(z, (0, NPAD - N))
    d2, s0p = _prep(xyz[:, 0], xyz[:, 1], xyz[:, 2], a0, a1, zp, embed)
    s = s0p[:N]
    d2 = d2[:, None]
    for p in params:
        w = _edge_filter(d2, p['We1'], p['be1'][None, :],
                         p['We2'], p['be2'][None, :])
        rn = _rn(s, p['Wn'], p['bn'][None, :])
        agg = _msg(rn, w, a0, a1)
        s = _update(agg[:N], agg[N:], s,
                    p['Wu1'], p['bu1'][None, :],
                    p['Wu2'], p['bu2'][None, :])
    return s


# speedup vs baseline: 5.3763x; 1.4088x over previous
import functools
import jax
import jax.numpy as jnp
from jax import lax
from jax.experimental import pallas as pl
from jax.experimental.pallas import tpu as pltpu
from jax.experimental.pallas import tpu_sc as plsc

N = 10000
E = 320000
F = 128
NG = 50
CUTOFF = 5.0
_WIDTH = CUTOFF / (NG - 1)
_COEFF = -0.5 / (_WIDTH * _WIDTH)
_LOG2 = 0.6931471805599453

NC, NS = 2, 16
NW = NC * NS
EPW = E // NW
EC = 40
NECH = EPW // EC
NPAD = 10240
ZPW = NPAD // NW
ZC = 64
NZCH = ZPW // ZC
RC = 40
NRCH = N // RC

_mesh = functools.partial(plsc.VectorSubcoreMesh,
                          core_axis_name="c", subcore_axis_name="s")
_SC_PARAMS = pltpu.CompilerParams(needs_layout_passes=False)


def _prep_body(xh, yh, zzh, a0h, a1h, zh, embh, d2h, s0h,
               xc, yc, zc, a0v, a1v, d2v, zidx, erows, sem):
    cid = lax.axis_index("c")
    sid = lax.axis_index("s")
    wid = sid * NC + cid

    pltpu.sync_copy(xh, xc)
    pltpu.sync_copy(yh, yc)
    pltpu.sync_copy(zzh, zc)
    ebase = wid * EPW
    pltpu.sync_copy(a0h.at[pl.ds(ebase, EPW)], a0v)
    pltpu.sync_copy(a1h.at[pl.ds(ebase, EPW)], a1v)

    def dist_step(i, carry):
        i0 = a0v[pl.ds(i * 16, 16)]
        i1 = a1v[pl.ds(i * 16, 16)]
        dx = plsc.load_gather(xc, [i0]) - plsc.load_gather(xc, [i1])
        dy = plsc.load_gather(yc, [i0]) - plsc.load_gather(yc, [i1])
        dz = plsc.load_gather(zc, [i0]) - plsc.load_gather(zc, [i1])
        d2v[pl.ds(i * 16, 16)] = dx * dx + dy * dy + dz * dz
        return carry

    lax.fori_loop(0, EPW // 16, dist_step, 0)
    pltpu.sync_copy(d2v, d2h.at[pl.ds(ebase, EPW)])

    nbase = wid * ZPW

    def emb_step(k, carry):
        off = nbase + k * ZC
        pltpu.sync_copy(zh.at[pl.ds(off, ZC)], zidx)
        pltpu.async_copy(embh.at[zidx], erows, sem).wait()
        pltpu.sync_copy(erows, s0h.at[pl.ds(off, ZC)])
        return carry

    lax.fori_loop(0, NZCH, emb_step, 0)


@jax.jit
def _prep(xcol, ycol, zcol, a0, a1, zp, embed):
    return pl.kernel(
        _prep_body,
        out_type=(jax.ShapeDtypeStruct((E,), jnp.float32),
                  jax.ShapeDtypeStruct((NPAD, F), jnp.float32)),
        mesh=_mesh(),
        scratch_types=[
            pltpu.VMEM((N,), jnp.float32),
            pltpu.VMEM((N,), jnp.float32),
            pltpu.VMEM((N,), jnp.float32),
            pltpu.VMEM((EPW,), jnp.int32),
            pltpu.VMEM((EPW,), jnp.int32),
            pltpu.VMEM((EPW,), jnp.float32),
            pltpu.VMEM((ZC,), jnp.int32),
            pltpu.VMEM((ZC, F), jnp.float32),
            pltpu.SemaphoreType.DMA,
        ],
        compiler_params=_SC_PARAMS,
    )(xcol, ycol, zcol, a0, a1, zp, embed)


def _msg_body(rnh, wh, a0h, a1h, aggh,
              aggs, a0v0, a0v1, a1v0, a1v1, r00, r01, r10, r11,
              wv0, wv1, vbuf, semi0, semi1, semg0, semg1):
    cid = lax.axis_index("c")
    sid = lax.axis_index("s")
    wid = sid * NC + cid
    a0v = (a0v0, a0v1)
    a1v = (a1v0, a1v1)
    r0 = (r00, r01)
    r1 = (r10, r11)
    wv = (wv0, wv1)
    semi = (semi0, semi1)
    semg = (semg0, semg1)
    zeros = jnp.zeros((16,), jnp.float32)

    def zrow(i, carry):
        for j in range(F // 16):
            vbuf[i, pl.ds(j * 16, 16)] = zeros
        return carry

    lax.fori_loop(0, RC, zrow, 0)

    def zchunk(i, carry):
        k = sid + i * NS

        @pl.when(k < NRCH)
        def _():
            pltpu.sync_copy(vbuf, aggs.at[pl.ds(k * RC, RC)])

        return carry

    lax.fori_loop(0, (NRCH + NS - 1) // NS, zchunk, 0)
    plsc.subcore_barrier()

    ebase = wid * EPW

    def _off(g):
        return ebase + g * EC

    def issue_in(g, b):
        pltpu.async_copy(a0h.at[pl.ds(_off(g), EC)], a0v[b], semi[b])
        pltpu.async_copy(a1h.at[pl.ds(_off(g), EC)], a1v[b], semi[b])
        pltpu.async_copy(wh.at[pl.ds(_off(g), EC)], wv[b], semi[b])

    def wait_in(g, b):
        pltpu.make_async_copy(a0h.at[pl.ds(_off(g), EC)], a0v[b], semi[b]).wait()
        pltpu.make_async_copy(a1h.at[pl.ds(_off(g), EC)], a1v[b], semi[b]).wait()
        pltpu.make_async_copy(wh.at[pl.ds(_off(g), EC)], wv[b], semi[b]).wait()

    def issue_g(b):
        pltpu.async_copy(rnh.at[a0v[b]], r0[b], semg[b])
        pltpu.async_copy(rnh.at[a1v[b]], r1[b], semg[b])

    def wait_g(b):
        pltpu.make_async_copy(rnh.at[a0v[b]], r0[b], semg[b]).wait()
        pltpu.make_async_copy(rnh.at[a1v[b]], r1[b], semg[b]).wait()

    def compute_scatter(b):
        def mul(i, c2):
            for j in range(F // 16):
                sl = pl.ds(j * 16, 16)
                ww = wv[b][i, sl]
                r0[b][i, sl] = r0[b][i, sl] * ww
                r1[b][i, sl] = r1[b][i, sl] * ww
            return c2

        lax.fori_loop(0, EC, mul, 0)
        pltpu.sync_copy(r0[b], aggs.at[a1v[b]], add=True)
        pltpu.sync_copy(r1[b], aggs.at[a0v[b]], add=True)

    issue_in(0, 0)
    wait_in(0, 0)
    issue_g(0)
    issue_in(1, 1)

    def outer(k, carry):
        for b in range(2):
            g = 2 * k + b
            nb = 1 - b
            wait_g(b)

            @pl.when(g + 1 < NECH)
            def _():
                wait_in(g + 1, nb)
                issue_g(nb)

            compute_scatter(b)

            @pl.when(g + 2 < NECH)
            def _():
                issue_in(g + 2, b)

        return carry

    lax.fori_loop(0, NECH // 2, outer, 0)
    plsc.subcore_barrier()

    def ochunk(i, carry):
        k = sid + i * NS

        @pl.when(k < NRCH)
        def _():
            r = k * RC
            pltpu.sync_copy(aggs.at[pl.ds(r, RC)],
                            aggh.at[pl.ds(cid * N + r, RC)])

        return carry

    lax.fori_loop(0, (NRCH + NS - 1) // NS, ochunk, 0)


@jax.jit
def _msg(rn, w, a0, a1):
    return pl.kernel(
        _msg_body,
        out_type=jax.ShapeDtypeStruct((2 * N, F), jnp.float32),
        mesh=_mesh(),
        scratch_types=[
            pltpu.VMEM_SHARED((N, F), jnp.float32),
            pltpu.VMEM((EC,), jnp.int32),
            pltpu.VMEM((EC,), jnp.int32),
            pltpu.VMEM((EC,), jnp.int32),
            pltpu.VMEM((EC,), jnp.int32),
            pltpu.VMEM((EC, F), jnp.float32),
            pltpu.VMEM((EC, F), jnp.float32),
            pltpu.VMEM((EC, F), jnp.float32),
            pltpu.VMEM((EC, F), jnp.float32),
            pltpu.VMEM((EC, F), jnp.float32),
            pltpu.VMEM((EC, F), jnp.float32),
            pltpu.VMEM((RC, F), jnp.float32),
            pltpu.SemaphoreType.DMA,
            pltpu.SemaphoreType.DMA,
            pltpu.SemaphoreType.DMA,
            pltpu.SemaphoreType.DMA,
        ],
        compiler_params=_SC_PARAMS,
    )(rn, w, a0, a1)


def _ssp(x):
    return jax.nn.softplus(x) - _LOG2


def _edge_filter_body(d2_ref, we1, be1, we2, be2, w_ref):
    e = jnp.sqrt(d2_ref[:])
    eb = e.shape[0]
    offs = lax.broadcasted_iota(jnp.int32, (1, NG), 1).astype(jnp.float32) * _WIDTH
    diff = jnp.broadcast_to(e, (eb, NG)) - offs
    g = jnp.exp(_COEFF * diff * diff)
    h = _ssp(jnp.dot(g, we1[:], preferred_element_type=jnp.float32) + be1[:])
    w_ref[:] = jnp.dot(h, we2[:], preferred_element_type=jnp.float32) + be2[:]


_EB = 2000


@jax.jit
def _edge_filter(d2, we1, be1, we2, be2):
    return pl.pallas_call(
        _edge_filter_body,
        grid=(E // _EB,),
        in_specs=[
            pl.BlockSpec((_EB, 1), lambda i: (i, 0)),
            pl.BlockSpec((NG, NG), lambda i: (0, 0)),
            pl.BlockSpec((1, NG), lambda i: (0, 0)),
            pl.BlockSpec((NG, F), lambda i: (0, 0)),
            pl.BlockSpec((1, F), lambda i: (0, 0)),
        ],
        out_specs=pl.BlockSpec((_EB, F), lambda i: (i, 0)),
        out_shape=jax.ShapeDtypeStruct((E, F), jnp.float32),
        compiler_params=pltpu.CompilerParams(
            dimension_semantics=("arbitrary",)),
    )(d2, we1, be1, we2, be2)


def _rn_body(s_ref, wn, bn, rn_ref):
    rn_ref[:] = jnp.dot(s_ref[:], wn[:],
                        preferred_element_type=jnp.float32) + bn[:]


_NB = 2000


@jax.jit
def _rn(s, wn, bn):
    return pl.pallas_call(
        _rn_body,
        grid=(N // _NB,),
        in_specs=[
            pl.BlockSpec((_NB, F), lambda i: (i, 0)),
            pl.BlockSpec((F, F), lambda i: (0, 0)),
            pl.BlockSpec((1, F), lambda i: (0, 0)),
        ],
        out_specs=pl.BlockSpec((_NB, F), lambda i: (i, 0)),
        out_shape=jax.ShapeDtypeStruct((N, F), jnp.float32),
        compiler_params=pltpu.CompilerParams(
            dimension_semantics=("arbitrary",)),
    )(s, wn, bn)


def _update_body(a0_ref, a1_ref, s_ref, wu1, bu1, wu2, bu2, out_ref):
    agg = a0_ref[:] + a1_ref[:]
    u = _ssp(jnp.dot(agg, wu1[:], preferred_element_type=jnp.float32) + bu1[:])
    u = jnp.dot(u, wu2[:], preferred_element_type=jnp.float32) + bu2[:]
    out_ref[:] = s_ref[:] + u


@jax.jit
def _update(agg0, agg1, s, wu1, bu1, wu2, bu2):
    return pl.pallas_call(
        _update_body,
        grid=(N // _NB,),
        in_specs=[
            pl.BlockSpec((_NB, F), lambda i: (i, 0)),
            pl.BlockSpec((_NB, F), lambda i: (i, 0)),
            pl.BlockSpec((_NB, F), lambda i: (i, 0)),
            pl.BlockSpec((F, F), lambda i: (0, 0)),
            pl.BlockSpec((1, F), lambda i: (0, 0)),
            pl.BlockSpec((F, F), lambda i: (0, 0)),
            pl.BlockSpec((1, F), lambda i: (0, 0)),
        ],
        out_specs=pl.BlockSpec((_NB, F), lambda i: (i, 0)),
        out_shape=jax.ShapeDtypeStruct((N, F), jnp.float32),
        compiler_params=pltpu.CompilerParams(
            dimension_semantics=("arbitrary",)),
    )(agg0, agg1, s, wu1, bu1, wu2, bu2)


def kernel(z, xyz, nbr_list, embed, params):
    a0 = nbr_list[:, 0]
    a1 = nbr_list[:, 1]
    zp = jnp.pad(z, (0, NPAD - N))
    d2, s0p = _prep(xyz[:, 0], xyz[:, 1], xyz[:, 2], a0, a1, zp, embed)
    s = s0p[:N]
    d2 = d2[:, None]
    for p in params:
        w = _edge_filter(d2, p['We1'], p['be1'][None, :],
                         p['We2'], p['be2'][None, :])
        rn = _rn(s, p['Wn'], p['bn'][None, :])
        agg = _msg(rn, w, a0, a1)
        s = _update(agg[:N], agg[N:], s,
                    p['Wu1'], p['bu1'][None, :],
                    p['Wu2'], p['bu2'][None, :])
    return s


# speedup vs baseline: 7.1595x; 1.3317x over previous
import functools
import jax
import jax.numpy as jnp
from jax import lax
from jax.experimental import pallas as pl
from jax.experimental.pallas import tpu as pltpu
from jax.experimental.pallas import tpu_sc as plsc

N = 10000
E = 320000
F = 128
NG = 50
CUTOFF = 5.0
_WIDTH = CUTOFF / (NG - 1)
_COEFF = -0.5 / (_WIDTH * _WIDTH)
_LOG2 = 0.6931471805599453

NC, NS = 2, 16
NW = NC * NS
EPW = E // NW
EC = 40
NECH = EPW // EC
NPAD = 10240
ZPW = NPAD // NW
ZC = 64
NZCH = ZPW // ZC
RC = 40
NRCH = N // RC

_mesh = functools.partial(plsc.VectorSubcoreMesh,
                          core_axis_name="c", subcore_axis_name="s")
_SC_PARAMS = pltpu.CompilerParams(needs_layout_passes=False)


def _prep_body(xh, yh, zzh, a0h, a1h, zh, embh, d2h, s0h,
               xc, yc, zc, a0v, a1v, d2v, zidx, erows, sem):
    cid = lax.axis_index("c")
    sid = lax.axis_index("s")
    wid = sid * NC + cid

    pltpu.sync_copy(xh, xc)
    pltpu.sync_copy(yh, yc)
    pltpu.sync_copy(zzh, zc)
    ebase = wid * EPW
    pltpu.sync_copy(a0h.at[pl.ds(ebase, EPW)], a0v)
    pltpu.sync_copy(a1h.at[pl.ds(ebase, EPW)], a1v)

    def dist_step(i, carry):
        i0 = a0v[pl.ds(i * 16, 16)]
        i1 = a1v[pl.ds(i * 16, 16)]
        dx = plsc.load_gather(xc, [i0]) - plsc.load_gather(xc, [i1])
        dy = plsc.load_gather(yc, [i0]) - plsc.load_gather(yc, [i1])
        dz = plsc.load_gather(zc, [i0]) - plsc.load_gather(zc, [i1])
        d2v[pl.ds(i * 16, 16)] = dx * dx + dy * dy + dz * dz
        return carry

    lax.fori_loop(0, EPW // 16, dist_step, 0)
    pltpu.sync_copy(d2v, d2h.at[pl.ds(ebase, EPW)])

    nbase = wid * ZPW

    def emb_step(k, carry):
        off = nbase + k * ZC
        pltpu.sync_copy(zh.at[pl.ds(off, ZC)], zidx)
        pltpu.async_copy(embh.at[zidx], erows, sem).wait()
        pltpu.sync_copy(erows, s0h.at[pl.ds(off, ZC)])
        return carry

    lax.fori_loop(0, NZCH, emb_step, 0)


@jax.jit
def _prep(xcol, ycol, zcol, a0, a1, zp, embed):
    return pl.kernel(
        _prep_body,
        out_type=(jax.ShapeDtypeStruct((E,), jnp.float32),
                  jax.ShapeDtypeStruct((NPAD, F), jnp.float32)),
        mesh=_mesh(),
        scratch_types=[
            pltpu.VMEM((N,), jnp.float32),
            pltpu.VMEM((N,), jnp.float32),
            pltpu.VMEM((N,), jnp.float32),
            pltpu.VMEM((EPW,), jnp.int32),
            pltpu.VMEM((EPW,), jnp.int32),
            pltpu.VMEM((EPW,), jnp.float32),
            pltpu.VMEM((ZC,), jnp.int32),
            pltpu.VMEM((ZC, F), jnp.float32),
            pltpu.SemaphoreType.DMA,
        ],
        compiler_params=_SC_PARAMS,
    )(xcol, ycol, zcol, a0, a1, zp, embed)


def _msg_body(rnh, wh, a0h, a1h, aggh,
              aggs, a0m, a1m, r00, r01, r10, r11,
              wv0, wv1, semi0, semi1, semg0, semg1, sems0, sems1):
    cid = lax.axis_index("c")
    sid = lax.axis_index("s")
    wid = sid * NC + cid
    r0 = (r00, r01)
    r1 = (r10, r11)
    wv = (wv0, wv1)
    semi = (semi0, semi1)
    semg = (semg0, semg1)
    sems = (sems0, sems1)
    zeros = jnp.zeros((16,), jnp.float32)

    pltpu.sync_copy(a0h.at[pl.ds(wid * EPW, EPW)], a0m)
    pltpu.sync_copy(a1h.at[pl.ds(wid * EPW, EPW)], a1m)

    def idx(m, g):
        return m.at[pl.ds(g * EC, EC)]

    def zrow(i, carry):
        for j in range(F // 16):
            r00[i, pl.ds(j * 16, 16)] = zeros
        return carry

    lax.fori_loop(0, EC, zrow, 0)

    def zchunk(i, carry):
        k = sid + i * NS

        @pl.when(k < NRCH)
        def _():
            pltpu.sync_copy(r00, aggs.at[pl.ds(k * RC, RC)])

        return carry

    lax.fori_loop(0, (NRCH + NS - 1) // NS, zchunk, 0)
    plsc.subcore_barrier()

    ebase = wid * EPW

    def issue_w(g, b):
        pltpu.async_copy(wh.at[pl.ds(ebase + g * EC, EC)], wv[b], semi[b])

    def wait_w(g, b):
        pltpu.make_async_copy(wh.at[pl.ds(ebase + g * EC, EC)], wv[b],
                              semi[b]).wait()

    def issue_g(g, b):
        pltpu.async_copy(rnh.at[idx(a0m, g)], r0[b], semg[b])
        pltpu.async_copy(rnh.at[idx(a1m, g)], r1[b], semg[b])

    def wait_g(g, b):
        pltpu.make_async_copy(rnh.at[idx(a0m, g)], r0[b], semg[b]).wait()
        pltpu.make_async_copy(rnh.at[idx(a1m, g)], r1[b], semg[b]).wait()

    def issue_s(g, b):
        pltpu.async_copy(r0[b], aggs.at[idx(a1m, g)], sems[b], add=True)
        pltpu.async_copy(r1[b], aggs.at[idx(a0m, g)], sems[b], add=True)

    def wait_s(g, b):
        pltpu.make_async_copy(r0[b], aggs.at[idx(a1m, g)], sems[b]).wait()
        pltpu.make_async_copy(r1[b], aggs.at[idx(a0m, g)], sems[b]).wait()

    def compute(b):
        def mul(i, c2):
            for j in range(F // 16):
                sl = pl.ds(j * 16, 16)
                ww = wv[b][i, sl]
                r0[b][i, sl] = r0[b][i, sl] * ww
                r1[b][i, sl] = r1[b][i, sl] * ww
            return c2

        lax.fori_loop(0, EC, mul, 0)

    issue_w(0, 0)
    issue_g(0, 0)
    issue_w(1, 1)

    def outer(k, carry):
        for b in range(2):
            g = 2 * k + b
            nb = 1 - b
            wait_g(g, b)
            wait_w(g, b)

            @pl.when(jnp.logical_and(g + 1 < NECH, g >= 1))
            def _():
                wait_s(g - 1, nb)

            @pl.when(g + 1 < NECH)
            def _():
                issue_g(g + 1, nb)

            compute(b)
            issue_s(g, b)

            @pl.when(g + 2 < NECH)
            def _():
                issue_w(g + 2, b)

        return carry

    lax.fori_loop(0, NECH // 2, outer, 0)
    wait_s(NECH - 2, 0)
    wait_s(NECH - 1, 1)
    plsc.subcore_barrier()

    def ochunk(i, carry):
        k = sid + i * NS

        @pl.when(k < NRCH)
        def _():
            r = k * RC
            pltpu.sync_copy(aggs.at[pl.ds(r, RC)],
                            aggh.at[pl.ds(cid * N + r, RC)])

        return carry

    lax.fori_loop(0, (NRCH + NS - 1) // NS, ochunk, 0)


@jax.jit
def _msg(rn, w, a0, a1):
    return pl.kernel(
        _msg_body,
        out_type=jax.ShapeDtypeStruct((2 * N, F), jnp.float32),
        mesh=_mesh(),
        scratch_types=[
            pltpu.VMEM_SHARED((N, F), jnp.float32),
            pltpu.VMEM((EPW,), jnp.int32),
            pltpu.VMEM((EPW,), jnp.int32),
            pltpu.VMEM((EC, F), jnp.float32),
            pltpu.VMEM((EC, F), jnp.float32),
            pltpu.VMEM((EC, F), jnp.float32),
            pltpu.VMEM((EC, F), jnp.float32),
            pltpu.VMEM((EC, F), jnp.float32),
            pltpu.VMEM((EC, F), jnp.float32),
            pltpu.SemaphoreType.DMA,
            pltpu.SemaphoreType.DMA,
            pltpu.SemaphoreType.DMA,
            pltpu.SemaphoreType.DMA,
            pltpu.SemaphoreType.DMA,
            pltpu.SemaphoreType.DMA,
        ],
        compiler_params=_SC_PARAMS,
    )(rn, w, a0, a1)


def _ssp(x):
    return jax.nn.softplus(x) - _LOG2


def _edge_filter_body(d2_ref, we1, be1, we2, be2, w_ref):
    e = jnp.sqrt(d2_ref[:])
    eb = e.shape[0]
    offs = lax.broadcasted_iota(jnp.int32, (1, NG), 1).astype(jnp.float32) * _WIDTH
    diff = jnp.broadcast_to(e, (eb, NG)) - offs
    g = jnp.exp(_COEFF * diff * diff)
    h = _ssp(jnp.dot(g, we1[:], preferred_element_type=jnp.float32) + be1[:])
    w_ref[:] = jnp.dot(h, we2[:], preferred_element_type=jnp.float32) + be2[:]


_EB = 2000


@jax.jit
def _edge_filter(d2, we1, be1, we2, be2):
    return pl.pallas_call(
        _edge_filter_body,
        grid=(E // _EB,),
        in_specs=[
            pl.BlockSpec((_EB, 1), lambda i: (i, 0)),
            pl.BlockSpec((NG, NG), lambda i: (0, 0)),
            pl.BlockSpec((1, NG), lambda i: (0, 0)),
            pl.BlockSpec((NG, F), lambda i: (0, 0)),
            pl.BlockSpec((1, F), lambda i: (0, 0)),
        ],
        out_specs=pl.BlockSpec((_EB, F), lambda i: (i, 0)),
        out_shape=jax.ShapeDtypeStruct((E, F), jnp.float32),
        compiler_params=pltpu.CompilerParams(
            dimension_semantics=("arbitrary",)),
    )(d2, we1, be1, we2, be2)


def _rn_body(s_ref, wn, bn, rn_ref):
    rn_ref[:] = jnp.dot(s_ref[:], wn[:],
                        preferred_element_type=jnp.float32) + bn[:]


_NB = 2000


@jax.jit
def _rn(s, wn, bn):
    return pl.pallas_call(
        _rn_body,
        grid=(N // _NB,),
        in_specs=[
            pl.BlockSpec((_NB, F), lambda i: (i, 0)),
            pl.BlockSpec((F, F), lambda i: (0, 0)),
            pl.BlockSpec((1, F), lambda i: (0, 0)),
        ],
        out_specs=pl.BlockSpec((_NB, F), lambda i: (i, 0)),
        out_shape=jax.ShapeDtypeStruct((N, F), jnp.float32),
        compiler_params=pltpu.CompilerParams(
            dimension_semantics=("arbitrary",)),
    )(s, wn, bn)


def _update_body(a0_ref, a1_ref, s_ref, wu1, bu1, wu2, bu2, out_ref):
    agg = a0_ref[:] + a1_ref[:]
    u = _ssp(jnp.dot(agg, wu1[:], preferred_element_type=jnp.float32) + bu1[:])
    u = jnp.dot(u, wu2[:], preferred_element_type=jnp.float32) + bu2[:]
    out_ref[:] = s_ref[:] + u


@jax.jit
def _update(agg0, agg1, s, wu1, bu1, wu2, bu2):
    return pl.pallas_call(
        _update_body,
        grid=(N // _NB,),
        in_specs=[
            pl.BlockSpec((_NB, F), lambda i: (i, 0)),
            pl.BlockSpec((_NB, F), lambda i: (i, 0)),
            pl.BlockSpec((_NB, F), lambda i: (i, 0)),
            pl.BlockSpec((F, F), lambda i: (0, 0)),
            pl.BlockSpec((1, F), lambda i: (0, 0)),
            pl.BlockSpec((F, F), lambda i: (0, 0)),
            pl.BlockSpec((1, F), lambda i: (0, 0)),
        ],
        out_specs=pl.BlockSpec((_NB, F), lambda i: (i, 0)),
        out_shape=jax.ShapeDtypeStruct((N, F), jnp.float32),
        compiler_params=pltpu.CompilerParams(
            dimension_semantics=("arbitrary",)),
    )(agg0, agg1, s, wu1, bu1, wu2, bu2)


def kernel(z, xyz, nbr_list, embed, params):
    a0 = nbr_list[:, 0]
    a1 = nbr_list[:, 1]
    zp = jnp.pad(z, (0, NPAD - N))
    d2, s0p = _prep(xyz[:, 0], xyz[:, 1], xyz[:, 2], a0, a1, zp, embed)
    s = s0p[:N]
    d2 = d2[:, None]
    for p in params:
        w = _edge_filter(d2, p['We1'], p['be1'][None, :],
                         p['We2'], p['be2'][None, :])
        rn = _rn(s, p['Wn'], p['bn'][None, :])
        agg = _msg(rn, w, a0, a1)
        s = _update(agg[:N], agg[N:], s,
                    p['Wu1'], p['bu1'][None, :],
                    p['Wu2'], p['bu2'][None, :])
    return s
